# Initial kernel scaffold; baseline (speedup 1.0000x reference)
import jax
import jax.numpy as jnp
from jax.experimental import pallas as pl


def kernel(log_w, particles, observation, A, C):
    raise NotImplementedError("write your pallas kernel here")


# speedup vs baseline: 1.9266x; 1.9266x over previous
import functools

import jax
import jax.numpy as jnp
from jax.experimental import pallas as pl
from jax.experimental.pallas import tpu as pltpu

N = 262144
D = 16
DO = 8
SIGMA = 0.5
TAU = 1.0
PACK = 8
RP = N // PACK

_ROWS = 512
_GRID = RP // _ROWS

_HALF_LOG_2PI = 0.5 * float(jnp.log(2.0 * jnp.pi))


def _dense_body(flag_ref, p_ref, rs_ref, eps_ref, lw_ref, abd_ref, cbd_ref,
                m16_ref, m8_ref, obs_ref, nxt_ref, newlw_ref, inc_ref):
    flag = flag_ref[0, 0] > 0
    pr = jnp.where(flag, rs_ref[...], p_ref[...])
    mean = jax.lax.dot(pr, abd_ref[...],
                       precision=jax.lax.Precision.HIGHEST)
    nxt = mean + jnp.float32(SIGMA) * eps_ref[...]
    nxt_ref[...] = nxt
    z = (nxt - mean) / jnp.float32(SIGMA)
    tz = (-0.5 * z * z - jnp.float32(jnp.log(SIGMA))) - jnp.float32(_HALF_LOG_2PI)
    t = jax.lax.dot(tz, m16_ref[...], precision=jax.lax.Precision.HIGHEST)
    em = jax.lax.dot(nxt, cbd_ref[...], precision=jax.lax.Precision.HIGHEST)
    ze = obs_ref[0:1, :] - em
    se = (-0.5 * ze * ze) - jnp.float32(_HALF_LOG_2PI)
    e = jax.lax.dot(se, m8_ref[...], precision=jax.lax.Precision.HIGHEST)
    inc = (t + e) - t
    inc_ref[...] = inc
    lwr = jnp.where(flag, jnp.full_like(inc, -jnp.log(jnp.float32(N))),
                    lw_ref[...])
    newlw_ref[...] = lwr + inc


@functools.partial(jax.jit, static_argnums=())
def kernel(log_w, particles, observation, A, C):
    key = jax.random.key(42)
    rk, pk = jax.random.split(key)
    u0 = jax.random.uniform(rk, (), dtype=jnp.float32)
    eps = jax.random.normal(pk, (N, D), dtype=jnp.float32)

    lw = log_w - jax.scipy.special.logsumexp(log_w)
    w = jnp.exp(lw)
    ess = 1.0 / jnp.sum(w * w)
    cum = jnp.cumsum(w)
    u = (jnp.arange(N, dtype=jnp.float32) + u0) / jnp.float32(N)
    idx = jnp.clip(jnp.searchsorted(cum, u), 0, N - 1)
    do_resample = ess < 0.5 * N
    ancestor_ix = jnp.where(do_resample, idx, jnp.arange(N))
    resampled = jnp.take(particles, idx, axis=0)

    p_pack = particles.reshape(RP, PACK * D)
    rs_pack = resampled.reshape(RP, PACK * D)
    eps_pack = eps.reshape(RP, PACK * D)
    lw_pack = lw.reshape(RP, PACK)
    flag = do_resample.astype(jnp.int32).reshape(1, 1)

    eye8 = jnp.eye(PACK, dtype=jnp.float32)
    abd = jnp.kron(eye8, A.T)
    cbd = jnp.kron(eye8, C.T)
    m16 = jnp.kron(eye8, jnp.ones((D, 1), jnp.float32))
    m8 = jnp.kron(eye8, jnp.ones((DO, 1), jnp.float32))
    obs_t = jnp.tile(observation, (PACK,)).reshape(1, PACK * DO)
    obs_t = jnp.broadcast_to(obs_t, (8, PACK * DO))

    grid = (_GRID,)
    row_spec = pl.BlockSpec((_ROWS, PACK * D), lambda i: (i, 0))
    out8_spec = pl.BlockSpec((_ROWS, PACK), lambda i: (i, 0))
    const_spec = lambda r, c: pl.BlockSpec((r, c), lambda i: (0, 0))
    nxt_pack, newlw_pack, inc_pack = pl.pallas_call(
        _dense_body,
        grid=grid,
        in_specs=[
            pl.BlockSpec(memory_space=pltpu.SMEM),
            row_spec,
            row_spec,
            row_spec,
            out8_spec,
            const_spec(PACK * D, PACK * D),
            const_spec(PACK * D, PACK * DO),
            const_spec(PACK * D, PACK),
            const_spec(PACK * DO, PACK),
            const_spec(8, PACK * DO),
        ],
        out_specs=[row_spec, out8_spec, out8_spec],
        out_shape=[
            jax.ShapeDtypeStruct((RP, PACK * D), jnp.float32),
            jax.ShapeDtypeStruct((RP, PACK), jnp.float32),
            jax.ShapeDtypeStruct((RP, PACK), jnp.float32),
        ],
    )(flag, p_pack, rs_pack, eps_pack, lw_pack, abd, cbd, m16, m8, obs_t)

    next_particles = nxt_pack.reshape(N, D)
    new_log_w = newlw_pack.reshape(N)
    inc_weight = inc_pack.reshape(N)
    return (new_log_w, next_particles, ess, ancestor_ix, inc_weight)


# speedup vs baseline: 15.1806x; 7.8793x over previous
import functools
import math

import jax
import jax.numpy as jnp
from jax import lax
from jax.experimental import pallas as pl
from jax.experimental.pallas import tpu as pltpu
from jax.experimental.pallas import tpu_sc as plsc

N = 262144
D = 16
DO = 8
SIGMA = 0.5
TAU = 1.0
PACK = 8
RP = N // PACK

_ROWS = 512
_GRID = RP // _ROWS

_AR = 256
_AGRID = (N // 128) // _AR

_NW = 32
_BW = N // _NW
_HPAD = N + 256
_HSLC = _HPAD // 16
_GSUB = 2048

_HALF_LOG_2PI = 0.5 * math.log(2.0 * math.pi)


def _fcount_body(u0_ref, cum_ref, w_ref, f_ref, ess_ref, flag_ref, acc_ref):
    i = pl.program_id(0)

    @pl.when(i == 0)
    def _():
        acc_ref[0, 0] = 0.0

    u0 = u0_ref[0, 0]
    cum = cum_ref[...]
    t = cum * jnp.float32(N)
    i0 = jnp.floor(t - u0)
    cnt = i0 - 1.0
    for dc in (-1.0, 0.0, 1.0):
        cnt = cnt + jnp.where((i0 + jnp.float32(dc)) + u0 <= t, 1.0, 0.0)
    f_ref[...] = jnp.clip(cnt, 0.0, jnp.float32(N)).astype(jnp.int32)

    w = w_ref[...]
    acc_ref[0, 0] += jnp.sum(w * w)

    @pl.when(i == _AGRID - 1)
    def _():
        ess = 1.0 / acc_ref[0, 0]
        ess_ref[...] = jnp.full((1, 1), ess, jnp.float32)
        flag_ref[...] = jnp.where(ess < jnp.float32(0.5 * N),
                                  jnp.full((1, 1), 1, jnp.int32),
                                  jnp.full((1, 1), 0, jnp.int32))


_sc_mesh = plsc.VectorSubcoreMesh(core_axis_name="c", subcore_axis_name="s")


@functools.partial(
    pl.kernel, mesh=_sc_mesh,
    out_type=jax.ShapeDtypeStruct((2, N), jnp.int32),
    scratch_types=[
        pltpu.VMEM((_BW,), jnp.int32),
        pltpu.VMEM((_BW,), jnp.int32),
        pltpu.VMEM((_HSLC,), jnp.int32),
        pltpu.VMEM_SHARED((_HPAD,), jnp.int32),
    ],
)
def _hist_sc(f_hbm, zeros_hbm, ones_hbm, h_hbm, f_v, ones_v, vbuf, h_sh):
    cid = lax.axis_index("c")
    sid = lax.axis_index("s")
    wid = sid * 2 + cid
    pltpu.sync_copy(zeros_hbm.at[pl.ds(sid * _HSLC, _HSLC)], vbuf)
    pltpu.sync_copy(vbuf, h_sh.at[pl.ds(sid * _HSLC, _HSLC)])
    plsc.subcore_barrier()
    pltpu.sync_copy(f_hbm.at[pl.ds(wid * _BW, _BW)], f_v)
    pltpu.sync_copy(ones_hbm, ones_v)
    pltpu.sync_copy(ones_v, h_sh.at[f_v], add=True)
    plsc.subcore_barrier()
    pltpu.sync_copy(h_sh.at[pl.ds(sid * (N // 16), N // 16)],
                    vbuf.at[pl.ds(0, N // 16)])
    pltpu.sync_copy(vbuf.at[pl.ds(0, N // 16)],
                    h_hbm.at[cid, pl.ds(sid * (N // 16), N // 16)])


def _scan_body(flag_ref, h_ref, u128_ref, l256_ref, idx_ref, anc_ref, car_ref):
    i = pl.program_id(0)

    @pl.when(i == 0)
    def _():
        car_ref[0, 0] = 0.0

    x = (h_ref[0] + h_ref[1]).astype(jnp.float32)
    lane_pref = jax.lax.dot(x, u128_ref[...],
                            precision=jax.lax.Precision.HIGHEST)
    row_tot = jnp.broadcast_to(lane_pref[:, 127:128], (_AR, 128))
    row_off = jax.lax.dot(l256_ref[...], row_tot,
                          precision=jax.lax.Precision.HIGHEST)
    idxf = lane_pref + row_off + car_ref[0, 0]
    car_ref[0, 0] += jnp.sum(x)
    idx = jnp.clip(idxf, 0.0, jnp.float32(N - 1)).astype(jnp.int32)
    idx_ref[...] = idx
    glob = (lax.broadcasted_iota(jnp.int32, (_AR, 128), 0) * 128
            + lax.broadcasted_iota(jnp.int32, (_AR, 128), 1)
            + i * (_AR * 128))
    anc_ref[...] = jnp.where(flag_ref[0, 0] > 0, idx, glob)


@functools.partial(
    pl.kernel, mesh=_sc_mesh,
    out_type=jax.ShapeDtypeStruct((N, D), jnp.float32),
    compiler_params=pltpu.CompilerParams(use_tc_tiling_on_sc=False),
    scratch_types=[
        pltpu.VMEM((_GSUB,), jnp.int32),
        pltpu.VMEM((_GSUB, D), jnp.float32),
        pltpu.SemaphoreType.DMA,
    ],
)
def _gather_sc(table_hbm, idx_hbm, out_hbm, idx_v, rows_v, sem):
    cid = lax.axis_index("c")
    sid = lax.axis_index("s")
    wid = sid * 2 + cid
    for g in range(_BW // _GSUB):
        base = wid * _BW + g * _GSUB
        pltpu.sync_copy(idx_hbm.at[pl.ds(base, _GSUB)], idx_v)
        pltpu.async_copy(table_hbm.at[idx_v], rows_v, sem).wait()
        pltpu.sync_copy(rows_v, out_hbm.at[pl.ds(base, _GSUB)])


def _dense_body(flag_ref, p_ref, rs_ref, eps_ref, lw_ref, abd_ref, cbd_ref,
                m16_ref, m8_ref, obs_ref, nxt_ref, newlw_ref, inc_ref):
    flag = flag_ref[0, 0] > 0
    pr = jnp.where(flag, rs_ref[...], p_ref[...])
    mean = jax.lax.dot(pr, abd_ref[...],
                       precision=jax.lax.Precision.HIGHEST)
    nxt = mean + jnp.float32(SIGMA) * eps_ref[...]
    nxt_ref[...] = nxt
    z = (nxt - mean) / jnp.float32(SIGMA)
    tz = (-0.5 * z * z - jnp.float32(math.log(SIGMA))) - jnp.float32(_HALF_LOG_2PI)
    t = jax.lax.dot(tz, m16_ref[...], precision=jax.lax.Precision.HIGHEST)
    em = jax.lax.dot(nxt, cbd_ref[...], precision=jax.lax.Precision.HIGHEST)
    ze = obs_ref[0:1, :] - em
    se = (-0.5 * ze * ze) - jnp.float32(_HALF_LOG_2PI)
    e = jax.lax.dot(se, m8_ref[...], precision=jax.lax.Precision.HIGHEST)
    inc = (t + e) - t
    inc_ref[...] = inc
    lwr = jnp.where(flag, jnp.full_like(inc, jnp.float32(-math.log(N))),
                    lw_ref[...])
    newlw_ref[...] = lwr + inc


@jax.jit
def kernel(log_w, particles, observation, A, C):
    key = jax.random.key(42)
    rk, pk = jax.random.split(key)
    u0 = jax.random.uniform(rk, (), dtype=jnp.float32)
    eps = jax.random.normal(pk, (N, D), dtype=jnp.float32)

    lw = log_w - jax.scipy.special.logsumexp(log_w)
    w = jnp.exp(lw)
    cum = jnp.cumsum(w)

    smem_scalar = pl.BlockSpec(memory_space=pltpu.SMEM)
    blk = pl.BlockSpec((_AR, 128), lambda i: (i, 0))
    one_spec = pl.BlockSpec((1, 1), lambda i: (0, 0))
    f2d, ess11, flag11 = pl.pallas_call(
        _fcount_body,
        grid=(_AGRID,),
        in_specs=[smem_scalar, blk, blk],
        out_specs=[blk, one_spec, one_spec],
        out_shape=[
            jax.ShapeDtypeStruct((N // 128, 128), jnp.int32),
            jax.ShapeDtypeStruct((1, 1), jnp.float32),
            jax.ShapeDtypeStruct((1, 1), jnp.int32),
        ],
        scratch_shapes=[pltpu.SMEM((1, 1), jnp.float32)],
    )(u0.reshape(1, 1), cum.reshape(N // 128, 128), w.reshape(N // 128, 128))

    h2 = _hist_sc(f2d.reshape(N),
                  jnp.zeros((_HPAD,), jnp.int32),
                  jnp.ones((_BW,), jnp.int32))

    u128 = jnp.triu(jnp.ones((128, 128), jnp.float32))
    l256 = jnp.tril(jnp.ones((_AR, _AR), jnp.float32), -1)
    idx2d, anc2d = pl.pallas_call(
        _scan_body,
        grid=(_AGRID,),
        in_specs=[
            smem_scalar,
            pl.BlockSpec((2, _AR, 128), lambda i: (0, i, 0)),
            pl.BlockSpec((128, 128), lambda i: (0, 0)),
            pl.BlockSpec((_AR, _AR), lambda i: (0, 0)),
        ],
        out_specs=[blk, blk],
        out_shape=[
            jax.ShapeDtypeStruct((N // 128, 128), jnp.int32),
            jax.ShapeDtypeStruct((N // 128, 128), jnp.int32),
        ],
        scratch_shapes=[pltpu.SMEM((1, 1), jnp.float32)],
    )(flag11, h2.reshape(2, N // 128, 128), u128, l256)

    resampled = _gather_sc(particles, idx2d.reshape(N))

    p_pack = particles.reshape(RP, PACK * D)
    rs_pack = resampled.reshape(RP, PACK * D)
    eps_pack = eps.reshape(RP, PACK * D)
    lw_pack = lw.reshape(RP, PACK)

    eye8 = jnp.eye(PACK, dtype=jnp.float32)
    abd = jnp.kron(eye8, A.T)
    cbd = jnp.kron(eye8, C.T)
    m16 = jnp.kron(eye8, jnp.ones((D, 1), jnp.float32))
    m8 = jnp.kron(eye8, jnp.ones((DO, 1), jnp.float32))
    obs_t = jnp.tile(observation, (PACK,)).reshape(1, PACK * DO)
    obs_t = jnp.broadcast_to(obs_t, (8, PACK * DO))

    row_spec = pl.BlockSpec((_ROWS, PACK * D), lambda i: (i, 0))
    out8_spec = pl.BlockSpec((_ROWS, PACK), lambda i: (i, 0))
    const_spec = lambda r, c: pl.BlockSpec((r, c), lambda i: (0, 0))
    nxt_pack, newlw_pack, inc_pack = pl.pallas_call(
        _dense_body,
        grid=(_GRID,),
        in_specs=[
            smem_scalar,
            row_spec,
            row_spec,
            row_spec,
            out8_spec,
            const_spec(PACK * D, PACK * D),
            const_spec(PACK * D, PACK * DO),
            const_spec(PACK * D, PACK),
            const_spec(PACK * DO, PACK),
            const_spec(8, PACK * DO),
        ],
        out_specs=[row_spec, out8_spec, out8_spec],
        out_shape=[
            jax.ShapeDtypeStruct((RP, PACK * D), jnp.float32),
            jax.ShapeDtypeStruct((RP, PACK), jnp.float32),
            jax.ShapeDtypeStruct((RP, PACK), jnp.float32),
        ],
    )(flag11, p_pack, rs_pack, eps_pack, lw_pack, abd, cbd, m16, m8, obs_t)

    next_particles = nxt_pack.reshape(N, D)
    new_log_w = newlw_pack.reshape(N)
    inc_weight = inc_pack.reshape(N)
    ess = ess11.reshape(())
    ancestor_ix = anc2d.reshape(N)
    return (new_log_w, next_particles, ess, ancestor_ix, inc_weight)


# speedup vs baseline: 17.5771x; 1.1579x over previous
import functools
import math

import jax
import jax.numpy as jnp
from jax import lax
from jax.experimental import pallas as pl
from jax.experimental.pallas import tpu as pltpu
from jax.experimental.pallas import tpu_sc as plsc

N = 262144
D = 16
DO = 8
SIGMA = 0.5
TAU = 1.0
PACK = 8
RP = N // PACK

_ROWS = 512
_GRID = RP // _ROWS

_AR = 256
_AGRID = (N // 128) // _AR

_NW = 32
_BW = N // _NW
_HPAD = N + 256
_HSLC = _HPAD // 16
_GSUB = 2048

_HALF_LOG_2PI = 0.5 * math.log(2.0 * math.pi)


def _fcount_body(u0_ref, cum_ref, w_ref, f_ref, ess_ref, flag_ref, acc_ref):
    i = pl.program_id(0)

    @pl.when(i == 0)
    def _():
        acc_ref[0, 0] = 0.0

    u0 = u0_ref[0, 0]
    cum = cum_ref[...]
    t = cum * jnp.float32(N)
    i0 = jnp.floor(t - u0)
    cnt = i0 - 1.0
    for dc in (-1.0, 0.0, 1.0):
        cnt = cnt + jnp.where((i0 + jnp.float32(dc)) + u0 <= t, 1.0, 0.0)
    f_ref[...] = jnp.clip(cnt, 0.0, jnp.float32(N)).astype(jnp.int32)

    w = w_ref[...]
    acc_ref[0, 0] += jnp.sum(w * w)

    @pl.when(i == _AGRID - 1)
    def _():
        ess = 1.0 / acc_ref[0, 0]
        ess_ref[...] = jnp.full((1, 1), ess, jnp.float32)
        flag_ref[...] = jnp.where(ess < jnp.float32(0.5 * N),
                                  jnp.full((1, 1), 1, jnp.int32),
                                  jnp.full((1, 1), 0, jnp.int32))


_sc_mesh = plsc.VectorSubcoreMesh(core_axis_name="c", subcore_axis_name="s")


@functools.partial(
    pl.kernel, mesh=_sc_mesh,
    out_type=jax.ShapeDtypeStruct((2, N), jnp.int32),
    scratch_types=[
        pltpu.VMEM((_BW,), jnp.int32),
        pltpu.VMEM((_BW,), jnp.int32),
        pltpu.VMEM((_HSLC,), jnp.int32),
        pltpu.VMEM_SHARED((_HPAD,), jnp.int32),
    ],
)
def _hist_sc(f_hbm, zeros_hbm, ones_hbm, h_hbm, f_v, ones_v, vbuf, h_sh):
    cid = lax.axis_index("c")
    sid = lax.axis_index("s")
    wid = sid * 2 + cid
    pltpu.sync_copy(zeros_hbm.at[pl.ds(sid * _HSLC, _HSLC)], vbuf)
    pltpu.sync_copy(vbuf, h_sh.at[pl.ds(sid * _HSLC, _HSLC)])
    plsc.subcore_barrier()
    pltpu.sync_copy(f_hbm.at[pl.ds(wid * _BW, _BW)], f_v)
    pltpu.sync_copy(ones_hbm, ones_v)
    pltpu.sync_copy(ones_v, h_sh.at[f_v], add=True)
    plsc.subcore_barrier()
    pltpu.sync_copy(h_sh.at[pl.ds(sid * (N // 16), N // 16)],
                    vbuf.at[pl.ds(0, N // 16)])
    pltpu.sync_copy(vbuf.at[pl.ds(0, N // 16)],
                    h_hbm.at[cid, pl.ds(sid * (N // 16), N // 16)])


def _scan_body(flag_ref, h_ref, u128_ref, l256_ref, anc_ref, car_ref):
    i = pl.program_id(0)

    @pl.when(i == 0)
    def _():
        car_ref[0, 0] = 0.0

    x = (h_ref[0] + h_ref[1]).astype(jnp.float32)
    lane_pref = jax.lax.dot(x, u128_ref[...],
                            precision=jax.lax.Precision.HIGHEST)
    row_tot = jnp.broadcast_to(lane_pref[:, 127:128], (_AR, 128))
    row_off = jax.lax.dot(l256_ref[...], row_tot,
                          precision=jax.lax.Precision.HIGHEST)
    idxf = lane_pref + row_off + car_ref[0, 0]
    car_ref[0, 0] += jnp.sum(x)
    idx = jnp.clip(idxf, 0.0, jnp.float32(N - 1)).astype(jnp.int32)
    glob = (lax.broadcasted_iota(jnp.int32, (_AR, 128), 0) * 128
            + lax.broadcasted_iota(jnp.int32, (_AR, 128), 1)
            + i * (_AR * 128))
    anc_ref[...] = jnp.where(flag_ref[0, 0] > 0, idx, glob)


@functools.partial(
    pl.kernel, mesh=_sc_mesh,
    out_type=jax.ShapeDtypeStruct((N, D), jnp.float32),
    compiler_params=pltpu.CompilerParams(use_tc_tiling_on_sc=False),
    scratch_types=[
        pltpu.VMEM((_GSUB,), jnp.int32),
        pltpu.VMEM((_GSUB, D), jnp.float32),
        pltpu.SemaphoreType.DMA,
    ],
)
def _gather_sc(table_hbm, idx_hbm, out_hbm, idx_v, rows_v, sem):
    cid = lax.axis_index("c")
    sid = lax.axis_index("s")
    wid = sid * 2 + cid
    for g in range(_BW // _GSUB):
        base = wid * _BW + g * _GSUB
        pltpu.sync_copy(idx_hbm.at[pl.ds(base, _GSUB)], idx_v)
        pltpu.async_copy(table_hbm.at[idx_v], rows_v, sem).wait()
        pltpu.sync_copy(rows_v, out_hbm.at[pl.ds(base, _GSUB)])


def _dense_body(flag_ref, rs_ref, eps_ref, lw_ref, abd_ref, cbd_ref,
                m16_ref, m8_ref, obs_ref, nxt_ref, newlw_ref, inc_ref):
    flag = flag_ref[0, 0] > 0
    pr = rs_ref[...]
    mean = jax.lax.dot(pr, abd_ref[...],
                       precision=jax.lax.Precision.HIGHEST)
    nxt = mean + jnp.float32(SIGMA) * eps_ref[...]
    nxt_ref[...] = nxt
    z = (nxt - mean) / jnp.float32(SIGMA)
    tz = (-0.5 * z * z - jnp.float32(math.log(SIGMA))) - jnp.float32(_HALF_LOG_2PI)
    t = jax.lax.dot(tz, m16_ref[...], precision=jax.lax.Precision.HIGHEST)
    em = jax.lax.dot(nxt, cbd_ref[...], precision=jax.lax.Precision.HIGHEST)
    ze = obs_ref[0:1, :] - em
    se = (-0.5 * ze * ze) - jnp.float32(_HALF_LOG_2PI)
    e = jax.lax.dot(se, m8_ref[...], precision=jax.lax.Precision.HIGHEST)
    inc = (t + e) - t
    inc_ref[...] = inc
    lwr = jnp.where(flag, jnp.full_like(inc, jnp.float32(-math.log(N))),
                    lw_ref[...])
    newlw_ref[...] = lwr + inc


@jax.jit
def kernel(log_w, particles, observation, A, C):
    key = jax.random.key(42)
    rk, pk = jax.random.split(key)
    u0 = jax.random.uniform(rk, (), dtype=jnp.float32)
    eps = jax.random.normal(pk, (N, D), dtype=jnp.float32)

    lw = log_w - jax.scipy.special.logsumexp(log_w)
    w = jnp.exp(lw)
    cum = jnp.cumsum(w)

    smem_scalar = pl.BlockSpec(memory_space=pltpu.SMEM)
    blk = pl.BlockSpec((_AR, 128), lambda i: (i, 0))
    one_spec = pl.BlockSpec((1, 1), lambda i: (0, 0))
    f2d, ess11, flag11 = pl.pallas_call(
        _fcount_body,
        grid=(_AGRID,),
        in_specs=[smem_scalar, blk, blk],
        out_specs=[blk, one_spec, one_spec],
        out_shape=[
            jax.ShapeDtypeStruct((N // 128, 128), jnp.int32),
            jax.ShapeDtypeStruct((1, 1), jnp.float32),
            jax.ShapeDtypeStruct((1, 1), jnp.int32),
        ],
        scratch_shapes=[pltpu.SMEM((1, 1), jnp.float32)],
    )(u0.reshape(1, 1), cum.reshape(N // 128, 128), w.reshape(N // 128, 128))

    h2 = _hist_sc(f2d.reshape(N),
                  jnp.zeros((_HPAD,), jnp.int32),
                  jnp.ones((_BW,), jnp.int32))

    u128 = jnp.triu(jnp.ones((128, 128), jnp.float32))
    l256 = jnp.tril(jnp.ones((_AR, _AR), jnp.float32), -1)
    anc2d = pl.pallas_call(
        _scan_body,
        grid=(_AGRID,),
        in_specs=[
            smem_scalar,
            pl.BlockSpec((2, _AR, 128), lambda i: (0, i, 0)),
            pl.BlockSpec((128, 128), lambda i: (0, 0)),
            pl.BlockSpec((_AR, _AR), lambda i: (0, 0)),
        ],
        out_specs=blk,
        out_shape=jax.ShapeDtypeStruct((N // 128, 128), jnp.int32),
        scratch_shapes=[pltpu.SMEM((1, 1), jnp.float32)],
    )(flag11, h2.reshape(2, N // 128, 128), u128, l256)

    resampled = _gather_sc(particles, anc2d.reshape(N))

    rs_pack = resampled.reshape(RP, PACK * D)
    eps_pack = eps.reshape(RP, PACK * D)
    lw_pack = lw.reshape(RP, PACK)

    eye8 = jnp.eye(PACK, dtype=jnp.float32)
    abd = jnp.kron(eye8, A.T)
    cbd = jnp.kron(eye8, C.T)
    m16 = jnp.kron(eye8, jnp.ones((D, 1), jnp.float32))
    m8 = jnp.kron(eye8, jnp.ones((DO, 1), jnp.float32))
    obs_t = jnp.tile(observation, (PACK,)).reshape(1, PACK * DO)
    obs_t = jnp.broadcast_to(obs_t, (8, PACK * DO))

    row_spec = pl.BlockSpec((_ROWS, PACK * D), lambda i: (i, 0))
    out8_spec = pl.BlockSpec((_ROWS, PACK), lambda i: (i, 0))
    const_spec = lambda r, c: pl.BlockSpec((r, c), lambda i: (0, 0))
    nxt_pack, newlw_pack, inc_pack = pl.pallas_call(
        _dense_body,
        grid=(_GRID,),
        in_specs=[
            smem_scalar,
            row_spec,
            row_spec,
            out8_spec,
            const_spec(PACK * D, PACK * D),
            const_spec(PACK * D, PACK * DO),
            const_spec(PACK * D, PACK),
            const_spec(PACK * DO, PACK),
            const_spec(8, PACK * DO),
        ],
        out_specs=[row_spec, out8_spec, out8_spec],
        out_shape=[
            jax.ShapeDtypeStruct((RP, PACK * D), jnp.float32),
            jax.ShapeDtypeStruct((RP, PACK), jnp.float32),
            jax.ShapeDtypeStruct((RP, PACK), jnp.float32),
        ],
    )(flag11, rs_pack, eps_pack, lw_pack, abd, cbd, m16, m8, obs_t)

    next_particles = nxt_pack.reshape(N, D)
    new_log_w = newlw_pack.reshape(N)
    inc_weight = inc_pack.reshape(N)
    ess = ess11.reshape(())
    ancestor_ix = anc2d.reshape(N)
    return (new_log_w, next_particles, ess, ancestor_ix, inc_weight)


# speedup vs baseline: 17.5908x; 1.0008x over previous
import functools
import math

import jax
import jax.numpy as jnp
from jax import lax
from jax.experimental import pallas as pl
from jax.experimental.pallas import tpu as pltpu
from jax.experimental.pallas import tpu_sc as plsc

N = 262144
D = 16
DO = 8
SIGMA = 0.5
TAU = 1.0
PACK = 8
RP = N // PACK

_ROWS = 512
_GRID = RP // _ROWS

_AR = 256
_AGRID = (N // 128) // _AR

_NW = 32
_BW = N // _NW
_HPAD = N + 256
_HSLC = _HPAD // 16
_GSUB = 4096

_HALF_LOG_2PI = 0.5 * math.log(2.0 * math.pi)


def _fcount_body(u0_ref, cum_ref, w_ref, f_ref, ess_ref, flag_ref, acc_ref):
    i = pl.program_id(0)

    @pl.when(i == 0)
    def _():
        acc_ref[0, 0] = 0.0

    u0 = u0_ref[0, 0]
    cum = cum_ref[...]
    t = cum * jnp.float32(N)
    i0 = jnp.floor(t - u0)
    cnt = i0 - 1.0
    for dc in (-1.0, 0.0, 1.0):
        cnt = cnt + jnp.where((i0 + jnp.float32(dc)) + u0 <= t, 1.0, 0.0)
    f_ref[...] = jnp.clip(cnt, 0.0, jnp.float32(N)).astype(jnp.int32)

    w = w_ref[...]
    acc_ref[0, 0] += jnp.sum(w * w)

    @pl.when(i == _AGRID - 1)
    def _():
        ess = 1.0 / acc_ref[0, 0]
        ess_ref[...] = jnp.full((1, 1), ess, jnp.float32)
        flag_ref[...] = jnp.where(ess < jnp.float32(0.5 * N),
                                  jnp.full((1, 1), 1, jnp.int32),
                                  jnp.full((1, 1), 0, jnp.int32))


_sc_mesh = plsc.VectorSubcoreMesh(core_axis_name="c", subcore_axis_name="s")


@functools.partial(
    pl.kernel, mesh=_sc_mesh,
    out_type=jax.ShapeDtypeStruct((2, N), jnp.int32),
    scratch_types=[
        pltpu.VMEM((_BW,), jnp.int32),
        pltpu.VMEM((_BW,), jnp.int32),
        pltpu.VMEM((_HSLC,), jnp.int32),
        pltpu.VMEM_SHARED((_HPAD,), jnp.int32),
    ],
)
def _hist_sc(f_hbm, zeros_hbm, ones_hbm, h_hbm, f_v, ones_v, vbuf, h_sh):
    cid = lax.axis_index("c")
    sid = lax.axis_index("s")
    wid = sid * 2 + cid
    pltpu.sync_copy(zeros_hbm.at[pl.ds(sid * _HSLC, _HSLC)], vbuf)
    pltpu.sync_copy(vbuf, h_sh.at[pl.ds(sid * _HSLC, _HSLC)])
    plsc.subcore_barrier()
    pltpu.sync_copy(f_hbm.at[pl.ds(wid * _BW, _BW)], f_v)
    pltpu.sync_copy(ones_hbm, ones_v)
    pltpu.sync_copy(ones_v, h_sh.at[f_v], add=True)
    plsc.subcore_barrier()
    pltpu.sync_copy(h_sh.at[pl.ds(sid * (N // 16), N // 16)],
                    vbuf.at[pl.ds(0, N // 16)])
    pltpu.sync_copy(vbuf.at[pl.ds(0, N // 16)],
                    h_hbm.at[cid, pl.ds(sid * (N // 16), N // 16)])


def _scan_body(flag_ref, h_ref, u128_ref, l256_ref, anc_ref, car_ref):
    i = pl.program_id(0)

    @pl.when(i == 0)
    def _():
        car_ref[0, 0] = 0.0

    x = (h_ref[0] + h_ref[1]).astype(jnp.float32)
    lane_pref = jax.lax.dot(x, u128_ref[...],
                            precision=jax.lax.Precision.HIGHEST)
    row_tot = jnp.broadcast_to(lane_pref[:, 127:128], (_AR, 128))
    row_off = jax.lax.dot(l256_ref[...], row_tot,
                          precision=jax.lax.Precision.HIGHEST)
    idxf = lane_pref + row_off + car_ref[0, 0]
    car_ref[0, 0] += jnp.sum(x)
    idx = jnp.clip(idxf, 0.0, jnp.float32(N - 1)).astype(jnp.int32)
    glob = (lax.broadcasted_iota(jnp.int32, (_AR, 128), 0) * 128
            + lax.broadcasted_iota(jnp.int32, (_AR, 128), 1)
            + i * (_AR * 128))
    anc_ref[...] = jnp.where(flag_ref[0, 0] > 0, idx, glob)


@functools.partial(
    pl.kernel, mesh=_sc_mesh,
    out_type=jax.ShapeDtypeStruct((N, D), jnp.float32),
    compiler_params=pltpu.CompilerParams(use_tc_tiling_on_sc=False),
    scratch_types=[
        pltpu.VMEM((_GSUB,), jnp.int32),
        pltpu.VMEM((_GSUB, D), jnp.float32),
        pltpu.SemaphoreType.DMA,
    ],
)
def _gather_sc(table_hbm, idx_hbm, out_hbm, idx_v, rows_v, sem):
    cid = lax.axis_index("c")
    sid = lax.axis_index("s")
    wid = sid * 2 + cid
    for g in range(_BW // _GSUB):
        base = wid * _BW + g * _GSUB
        pltpu.sync_copy(idx_hbm.at[pl.ds(base, _GSUB)], idx_v)
        pltpu.async_copy(table_hbm.at[idx_v], rows_v, sem).wait()
        pltpu.sync_copy(rows_v, out_hbm.at[pl.ds(base, _GSUB)])


def _dense_body(flag_ref, rs_ref, eps_ref, lw_ref, abd_ref, cbd_ref,
                m16_ref, m8_ref, obs_ref, nxt_ref, newlw_ref, inc_ref):
    flag = flag_ref[0, 0] > 0
    pr = rs_ref[...]
    mean = jax.lax.dot(pr, abd_ref[...],
                       precision=jax.lax.Precision.HIGHEST)
    nxt = mean + jnp.float32(SIGMA) * eps_ref[...]
    nxt_ref[...] = nxt
    z = (nxt - mean) / jnp.float32(SIGMA)
    tz = (-0.5 * z * z - jnp.float32(math.log(SIGMA))) - jnp.float32(_HALF_LOG_2PI)
    t = jax.lax.dot(tz, m16_ref[...], precision=jax.lax.Precision.HIGHEST)
    em = jax.lax.dot(nxt, cbd_ref[...], precision=jax.lax.Precision.HIGHEST)
    ze = obs_ref[0:1, :] - em
    se = (-0.5 * ze * ze) - jnp.float32(_HALF_LOG_2PI)
    e = jax.lax.dot(se, m8_ref[...], precision=jax.lax.Precision.HIGHEST)
    inc = (t + e) - t
    inc_ref[...] = inc
    lwr = jnp.where(flag, jnp.full_like(inc, jnp.float32(-math.log(N))),
                    lw_ref[...])
    newlw_ref[...] = lwr + inc


@jax.jit
def kernel(log_w, particles, observation, A, C):
    key = jax.random.key(42)
    rk, pk = jax.random.split(key)
    u0 = jax.random.uniform(rk, (), dtype=jnp.float32)
    eps = jax.random.normal(pk, (N, D), dtype=jnp.float32)

    lw = log_w - jax.scipy.special.logsumexp(log_w)
    w = jnp.exp(lw)
    cum = jnp.cumsum(w)

    smem_scalar = pl.BlockSpec(memory_space=pltpu.SMEM)
    blk = pl.BlockSpec((_AR, 128), lambda i: (i, 0))
    one_spec = pl.BlockSpec((1, 1), lambda i: (0, 0))
    f2d, ess11, flag11 = pl.pallas_call(
        _fcount_body,
        grid=(_AGRID,),
        in_specs=[smem_scalar, blk, blk],
        out_specs=[blk, one_spec, one_spec],
        out_shape=[
            jax.ShapeDtypeStruct((N // 128, 128), jnp.int32),
            jax.ShapeDtypeStruct((1, 1), jnp.float32),
            jax.ShapeDtypeStruct((1, 1), jnp.int32),
        ],
        scratch_shapes=[pltpu.SMEM((1, 1), jnp.float32)],
    )(u0.reshape(1, 1), cum.reshape(N // 128, 128), w.reshape(N // 128, 128))

    h2 = _hist_sc(f2d.reshape(N),
                  jnp.zeros((_HPAD,), jnp.int32),
                  jnp.ones((_BW,), jnp.int32))

    u128 = jnp.triu(jnp.ones((128, 128), jnp.float32))
    l256 = jnp.tril(jnp.ones((_AR, _AR), jnp.float32), -1)
    anc2d = pl.pallas_call(
        _scan_body,
        grid=(_AGRID,),
        in_specs=[
            smem_scalar,
            pl.BlockSpec((2, _AR, 128), lambda i: (0, i, 0)),
            pl.BlockSpec((128, 128), lambda i: (0, 0)),
            pl.BlockSpec((_AR, _AR), lambda i: (0, 0)),
        ],
        out_specs=blk,
        out_shape=jax.ShapeDtypeStruct((N // 128, 128), jnp.int32),
        scratch_shapes=[pltpu.SMEM((1, 1), jnp.float32)],
    )(flag11, h2.reshape(2, N // 128, 128), u128, l256)

    resampled = _gather_sc(particles, anc2d.reshape(N))

    rs_pack = resampled.reshape(RP, PACK * D)
    eps_pack = eps.reshape(RP, PACK * D)
    lw_pack = lw.reshape(RP, PACK)

    eye8 = jnp.eye(PACK, dtype=jnp.float32)
    abd = jnp.kron(eye8, A.T)
    cbd = jnp.kron(eye8, C.T)
    m16 = jnp.kron(eye8, jnp.ones((D, 1), jnp.float32))
    m8 = jnp.kron(eye8, jnp.ones((DO, 1), jnp.float32))
    obs_t = jnp.tile(observation, (PACK,)).reshape(1, PACK * DO)
    obs_t = jnp.broadcast_to(obs_t, (8, PACK * DO))

    row_spec = pl.BlockSpec((_ROWS, PACK * D), lambda i: (i, 0))
    out8_spec = pl.BlockSpec((_ROWS, PACK), lambda i: (i, 0))
    const_spec = lambda r, c: pl.BlockSpec((r, c), lambda i: (0, 0))
    nxt_pack, newlw_pack, inc_pack = pl.pallas_call(
        _dense_body,
        grid=(_GRID,),
        in_specs=[
            smem_scalar,
            row_spec,
            row_spec,
            out8_spec,
            const_spec(PACK * D, PACK * D),
            const_spec(PACK * D, PACK * DO),
            const_spec(PACK * D, PACK),
            const_spec(PACK * DO, PACK),
            const_spec(8, PACK * DO),
        ],
        out_specs=[row_spec, out8_spec, out8_spec],
        out_shape=[
            jax.ShapeDtypeStruct((RP, PACK * D), jnp.float32),
            jax.ShapeDtypeStruct((RP, PACK), jnp.float32),
            jax.ShapeDtypeStruct((RP, PACK), jnp.float32),
        ],
    )(flag11, rs_pack, eps_pack, lw_pack, abd, cbd, m16, m8, obs_t)

    next_particles = nxt_pack.reshape(N, D)
    new_log_w = newlw_pack.reshape(N)
    inc_weight = inc_pack.reshape(N)
    ess = ess11.reshape(())
    ancestor_ix = anc2d.reshape(N)
    return (new_log_w, next_particles, ess, ancestor_ix, inc_weight)


# speedup vs baseline: 18.7370x; 1.0652x over previous
import functools
import math

import jax
import jax.numpy as jnp
from jax import lax
from jax.experimental import pallas as pl
from jax.experimental.pallas import tpu as pltpu
from jax.experimental.pallas import tpu_sc as plsc

N = 262144
D = 16
DO = 8
SIGMA = 0.5
TAU = 1.0
PACK = 8
RP = N // PACK

_ROWS = 512
_GRID = RP // _ROWS

_AR = 256
_AGRID = (N // 128) // _AR

_NW = 32
_BW = N // _NW
_HPAD = N + 256
_HSLC = _HPAD // 16
_GSUB = 4096

_HALF_LOG_2PI = 0.5 * math.log(2.0 * math.pi)


def _fcount_body(u0_ref, cum_ref, w_ref, f_ref, ess_ref, flag_ref, acc_ref):
    i = pl.program_id(0)

    @pl.when(i == 0)
    def _():
        acc_ref[0, 0] = 0.0

    u0 = u0_ref[0, 0]
    cum = cum_ref[...]
    t = cum * jnp.float32(N)
    i0 = jnp.floor(t - u0)
    cnt = i0 - 1.0
    for dc in (-1.0, 0.0, 1.0):
        cnt = cnt + jnp.where((i0 + jnp.float32(dc)) + u0 <= t, 1.0, 0.0)
    f_ref[...] = jnp.clip(cnt, 0.0, jnp.float32(N)).astype(jnp.int32)

    w = w_ref[...]
    acc_ref[0, 0] += jnp.sum(w * w)

    @pl.when(i == _AGRID - 1)
    def _():
        ess = 1.0 / acc_ref[0, 0]
        ess_ref[...] = jnp.full((1, 1), ess, jnp.float32)
        flag_ref[...] = jnp.where(ess < jnp.float32(0.5 * N),
                                  jnp.full((1, 1), 1, jnp.int32),
                                  jnp.full((1, 1), 0, jnp.int32))


_sc_mesh = plsc.VectorSubcoreMesh(core_axis_name="c", subcore_axis_name="s")


@functools.partial(
    pl.kernel, mesh=_sc_mesh,
    out_type=jax.ShapeDtypeStruct((2, N), jnp.int32),
    scratch_types=[
        pltpu.VMEM((_BW,), jnp.int32),
        pltpu.VMEM((_BW,), jnp.int32),
        pltpu.VMEM((_HSLC,), jnp.int32),
        pltpu.VMEM_SHARED((_HPAD,), jnp.int32),
    ],
)
def _hist_sc(f_hbm, zeros_hbm, ones_hbm, h_hbm, f_v, ones_v, vbuf, h_sh):
    cid = lax.axis_index("c")
    sid = lax.axis_index("s")
    wid = sid * 2 + cid
    pltpu.sync_copy(zeros_hbm.at[pl.ds(sid * _HSLC, _HSLC)], vbuf)
    pltpu.sync_copy(vbuf, h_sh.at[pl.ds(sid * _HSLC, _HSLC)])
    plsc.subcore_barrier()
    pltpu.sync_copy(f_hbm.at[pl.ds(wid * _BW, _BW)], f_v)
    pltpu.sync_copy(ones_hbm, ones_v)
    pltpu.sync_copy(ones_v, h_sh.at[f_v], add=True)
    plsc.subcore_barrier()
    pltpu.sync_copy(h_sh.at[pl.ds(sid * (N // 16), N // 16)],
                    vbuf.at[pl.ds(0, N // 16)])
    pltpu.sync_copy(vbuf.at[pl.ds(0, N // 16)],
                    h_hbm.at[cid, pl.ds(sid * (N // 16), N // 16)])


def _scan_body(flag_ref, h_ref, u128_ref, l256_ref, anc_ref, car_ref):
    i = pl.program_id(0)

    @pl.when(i == 0)
    def _():
        car_ref[0, 0] = 0.0

    x = (h_ref[0] + h_ref[1]).astype(jnp.float32)
    lane_pref = jax.lax.dot(x, u128_ref[...],
                            precision=jax.lax.Precision.HIGHEST)
    row_tot = jnp.broadcast_to(lane_pref[:, 127:128], (_AR, 128))
    row_off = jax.lax.dot(l256_ref[...], row_tot,
                          precision=jax.lax.Precision.HIGHEST)
    idxf = lane_pref + row_off + car_ref[0, 0]
    car_ref[0, 0] += jnp.sum(x)
    idx = jnp.clip(idxf, 0.0, jnp.float32(N - 1)).astype(jnp.int32)
    glob = (lax.broadcasted_iota(jnp.int32, (_AR, 128), 0) * 128
            + lax.broadcasted_iota(jnp.int32, (_AR, 128), 1)
            + i * (_AR * 128))
    anc_ref[...] = jnp.where(flag_ref[0, 0] > 0, idx, glob)


@functools.partial(
    pl.kernel, mesh=_sc_mesh,
    out_type=jax.ShapeDtypeStruct((N, D), jnp.float32),
    compiler_params=pltpu.CompilerParams(use_tc_tiling_on_sc=False),
    scratch_types=[
        pltpu.VMEM((_GSUB,), jnp.int32),
        pltpu.VMEM((_GSUB, D), jnp.float32),
        pltpu.SemaphoreType.DMA,
    ],
)
def _gather_sc(table_hbm, idx_hbm, out_hbm, idx_v, rows_v, sem):
    cid = lax.axis_index("c")
    sid = lax.axis_index("s")
    wid = sid * 2 + cid
    for g in range(_BW // _GSUB):
        base = wid * _BW + g * _GSUB
        pltpu.sync_copy(idx_hbm.at[pl.ds(base, _GSUB)], idx_v)
        pltpu.async_copy(table_hbm.at[idx_v], rows_v, sem).wait()
        pltpu.sync_copy(rows_v, out_hbm.at[pl.ds(base, _GSUB)])


def _dense_body(flag_ref, rs_ref, eps_ref, lw_ref, abd_ref, cbd_ref,
                m16_ref, m8_ref, obs_ref, nxt_ref, newlw_ref, inc_ref):
    flag = flag_ref[0, 0] > 0
    pr = rs_ref[...]
    mean = jax.lax.dot(pr, abd_ref[...],
                       precision=jax.lax.Precision.HIGHEST)
    nxt = mean + jnp.float32(SIGMA) * eps_ref[...]
    nxt_ref[...] = nxt
    z = (nxt - mean) / jnp.float32(SIGMA)
    tz = (-0.5 * z * z - jnp.float32(math.log(SIGMA))) - jnp.float32(_HALF_LOG_2PI)
    t = jax.lax.dot(tz, m16_ref[...], precision=jax.lax.Precision.HIGHEST)
    em = jax.lax.dot(nxt, cbd_ref[...], precision=jax.lax.Precision.HIGHEST)
    ze = obs_ref[0:1, :] - em
    se = (-0.5 * ze * ze) - jnp.float32(_HALF_LOG_2PI)
    e = jax.lax.dot(se, m8_ref[...], precision=jax.lax.Precision.HIGHEST)
    inc = (t + e) - t
    inc_ref[...] = inc
    lwr = jnp.where(flag, jnp.full_like(inc, jnp.float32(-math.log(N))),
                    lw_ref[...])
    newlw_ref[...] = lwr + inc


@jax.jit
def kernel(log_w, particles, observation, A, C):
    key = jax.random.key(42)
    rk, pk = jax.random.split(key)
    u0 = jax.random.uniform(rk, (), dtype=jnp.float32)
    eps_pack = jax.random.normal(pk, (RP, PACK * D), dtype=jnp.float32)

    lw = log_w - jax.scipy.special.logsumexp(log_w)
    w = jnp.exp(lw)
    cum = jnp.cumsum(w)

    smem_scalar = pl.BlockSpec(memory_space=pltpu.SMEM)
    blk = pl.BlockSpec((_AR, 128), lambda i: (i, 0))
    one_spec = pl.BlockSpec((1, 1), lambda i: (0, 0))
    f2d, ess11, flag11 = pl.pallas_call(
        _fcount_body,
        grid=(_AGRID,),
        in_specs=[smem_scalar, blk, blk],
        out_specs=[blk, one_spec, one_spec],
        out_shape=[
            jax.ShapeDtypeStruct((N // 128, 128), jnp.int32),
            jax.ShapeDtypeStruct((1, 1), jnp.float32),
            jax.ShapeDtypeStruct((1, 1), jnp.int32),
        ],
        scratch_shapes=[pltpu.SMEM((1, 1), jnp.float32)],
    )(u0.reshape(1, 1), cum.reshape(N // 128, 128), w.reshape(N // 128, 128))

    h2 = _hist_sc(f2d.reshape(N),
                  jnp.zeros((_HPAD,), jnp.int32),
                  jnp.ones((_BW,), jnp.int32))

    u128 = jnp.triu(jnp.ones((128, 128), jnp.float32))
    l256 = jnp.tril(jnp.ones((_AR, _AR), jnp.float32), -1)
    anc2d = pl.pallas_call(
        _scan_body,
        grid=(_AGRID,),
        in_specs=[
            smem_scalar,
            pl.BlockSpec((2, _AR, 128), lambda i: (0, i, 0)),
            pl.BlockSpec((128, 128), lambda i: (0, 0)),
            pl.BlockSpec((_AR, _AR), lambda i: (0, 0)),
        ],
        out_specs=blk,
        out_shape=jax.ShapeDtypeStruct((N // 128, 128), jnp.int32),
        scratch_shapes=[pltpu.SMEM((1, 1), jnp.float32)],
    )(flag11, h2.reshape(2, N // 128, 128), u128, l256)

    resampled = _gather_sc(particles, anc2d.reshape(N))

    rs_pack = resampled.reshape(RP, PACK * D)
    lw_pack = lw.reshape(RP, PACK)

    eye8 = jnp.eye(PACK, dtype=jnp.float32)
    abd = jnp.kron(eye8, A.T)
    cbd = jnp.kron(eye8, C.T)
    m16 = jnp.kron(eye8, jnp.ones((D, 1), jnp.float32))
    m8 = jnp.kron(eye8, jnp.ones((DO, 1), jnp.float32))
    obs_t = jnp.tile(observation, (PACK,)).reshape(1, PACK * DO)
    obs_t = jnp.broadcast_to(obs_t, (8, PACK * DO))

    row_spec = pl.BlockSpec((_ROWS, PACK * D), lambda i: (i, 0))
    out8_spec = pl.BlockSpec((_ROWS, PACK), lambda i: (i, 0))
    const_spec = lambda r, c: pl.BlockSpec((r, c), lambda i: (0, 0))
    nxt_pack, newlw_pack, inc_pack = pl.pallas_call(
        _dense_body,
        grid=(_GRID,),
        in_specs=[
            smem_scalar,
            row_spec,
            row_spec,
            out8_spec,
            const_spec(PACK * D, PACK * D),
            const_spec(PACK * D, PACK * DO),
            const_spec(PACK * D, PACK),
            const_spec(PACK * DO, PACK),
            const_spec(8, PACK * DO),
        ],
        out_specs=[row_spec, out8_spec, out8_spec],
        out_shape=[
            jax.ShapeDtypeStruct((RP, PACK * D), jnp.float32),
            jax.ShapeDtypeStruct((RP, PACK), jnp.float32),
            jax.ShapeDtypeStruct((RP, PACK), jnp.float32),
        ],
    )(flag11, rs_pack, eps_pack, lw_pack, abd, cbd, m16, m8, obs_t)

    next_particles = nxt_pack.reshape(N, D)
    new_log_w = newlw_pack.reshape(N)
    inc_weight = inc_pack.reshape(N)
    ess = ess11.reshape(())
    ancestor_ix = anc2d.reshape(N)
    return (new_log_w, next_particles, ess, ancestor_ix, inc_weight)


# speedup vs baseline: 19.2831x; 1.0291x over previous
import functools
import math

import jax
import jax.numpy as jnp
from jax import lax
from jax.experimental import pallas as pl
from jax.experimental.pallas import tpu as pltpu
from jax.experimental.pallas import tpu_sc as plsc

N = 262144
D = 16
DO = 8
SIGMA = 0.5
TAU = 1.0
PACK = 8
RP = N // PACK

_ROWS = 1024
_GRID = RP // _ROWS

_AR = 256
_AGRID = (N // 128) // _AR

_NW = 32
_BW = N // _NW
_HPAD = N + 256
_HSLC = _HPAD // 16
_GSUB = 4096

_HALF_LOG_2PI = 0.5 * math.log(2.0 * math.pi)


def _fcount_body(u0_ref, cum_ref, w_ref, f_ref, ess_ref, flag_ref, acc_ref):
    i = pl.program_id(0)

    @pl.when(i == 0)
    def _():
        acc_ref[0, 0] = 0.0

    u0 = u0_ref[0, 0]
    cum = cum_ref[...]
    t = cum * jnp.float32(N)
    i0 = jnp.floor(t - u0)
    cnt = i0 - 1.0
    for dc in (-1.0, 0.0, 1.0):
        cnt = cnt + jnp.where((i0 + jnp.float32(dc)) + u0 <= t, 1.0, 0.0)
    f_ref[...] = jnp.clip(cnt, 0.0, jnp.float32(N)).astype(jnp.int32)

    w = w_ref[...]
    acc_ref[0, 0] += jnp.sum(w * w)

    @pl.when(i == _AGRID - 1)
    def _():
        ess = 1.0 / acc_ref[0, 0]
        ess_ref[...] = jnp.full((1, 1), ess, jnp.float32)
        flag_ref[...] = jnp.where(ess < jnp.float32(0.5 * N),
                                  jnp.full((1, 1), 1, jnp.int32),
                                  jnp.full((1, 1), 0, jnp.int32))


_sc_mesh = plsc.VectorSubcoreMesh(core_axis_name="c", subcore_axis_name="s")


@functools.partial(
    pl.kernel, mesh=_sc_mesh,
    out_type=jax.ShapeDtypeStruct((2, N), jnp.int32),
    scratch_types=[
        pltpu.VMEM((_BW,), jnp.int32),
        pltpu.VMEM((_BW,), jnp.int32),
        pltpu.VMEM((_HSLC,), jnp.int32),
        pltpu.VMEM_SHARED((_HPAD,), jnp.int32),
    ],
)
def _hist_sc(f_hbm, zeros_hbm, ones_hbm, h_hbm, f_v, ones_v, vbuf, h_sh):
    cid = lax.axis_index("c")
    sid = lax.axis_index("s")
    wid = sid * 2 + cid
    pltpu.sync_copy(zeros_hbm.at[pl.ds(sid * _HSLC, _HSLC)], vbuf)
    pltpu.sync_copy(vbuf, h_sh.at[pl.ds(sid * _HSLC, _HSLC)])
    plsc.subcore_barrier()
    pltpu.sync_copy(f_hbm.at[pl.ds(wid * _BW, _BW)], f_v)
    pltpu.sync_copy(ones_hbm, ones_v)
    pltpu.sync_copy(ones_v, h_sh.at[f_v], add=True)
    plsc.subcore_barrier()
    pltpu.sync_copy(h_sh.at[pl.ds(sid * (N // 16), N // 16)],
                    vbuf.at[pl.ds(0, N // 16)])
    pltpu.sync_copy(vbuf.at[pl.ds(0, N // 16)],
                    h_hbm.at[cid, pl.ds(sid * (N // 16), N // 16)])


def _scan_body(flag_ref, h_ref, u128_ref, l256_ref, anc_ref, car_ref):
    i = pl.program_id(0)

    @pl.when(i == 0)
    def _():
        car_ref[0, 0] = 0.0

    x = (h_ref[0] + h_ref[1]).astype(jnp.float32)
    lane_pref = jax.lax.dot(x, u128_ref[...],
                            precision=jax.lax.Precision.HIGHEST)
    row_tot = jnp.broadcast_to(lane_pref[:, 127:128], (_AR, 128))
    row_off = jax.lax.dot(l256_ref[...], row_tot,
                          precision=jax.lax.Precision.HIGHEST)
    idxf = lane_pref + row_off + car_ref[0, 0]
    car_ref[0, 0] += jnp.sum(x)
    idx = jnp.clip(idxf, 0.0, jnp.float32(N - 1)).astype(jnp.int32)
    glob = (lax.broadcasted_iota(jnp.int32, (_AR, 128), 0) * 128
            + lax.broadcasted_iota(jnp.int32, (_AR, 128), 1)
            + i * (_AR * 128))
    anc_ref[...] = jnp.where(flag_ref[0, 0] > 0, idx, glob)


@functools.partial(
    pl.kernel, mesh=_sc_mesh,
    out_type=jax.ShapeDtypeStruct((N, D), jnp.float32),
    compiler_params=pltpu.CompilerParams(use_tc_tiling_on_sc=False),
    scratch_types=[
        pltpu.VMEM((_GSUB,), jnp.int32),
        pltpu.VMEM((_GSUB, D), jnp.float32),
        pltpu.SemaphoreType.DMA,
    ],
)
def _gather_sc(table_hbm, idx_hbm, out_hbm, idx_v, rows_v, sem):
    cid = lax.axis_index("c")
    sid = lax.axis_index("s")
    wid = sid * 2 + cid
    for g in range(_BW // _GSUB):
        base = wid * _BW + g * _GSUB
        pltpu.sync_copy(idx_hbm.at[pl.ds(base, _GSUB)], idx_v)
        pltpu.async_copy(table_hbm.at[idx_v], rows_v, sem).wait()
        pltpu.sync_copy(rows_v, out_hbm.at[pl.ds(base, _GSUB)])


def _dense_body(flag_ref, rs_ref, eps_ref, lw_ref, abd_ref, cbd_ref,
                m16_ref, m8_ref, obs_ref, nxt_ref, newlw_ref, inc_ref):
    flag = flag_ref[0, 0] > 0
    pr = rs_ref[...]
    mean = jax.lax.dot(pr, abd_ref[...],
                       precision=jax.lax.Precision.HIGHEST)
    nxt = mean + jnp.float32(SIGMA) * eps_ref[...]
    nxt_ref[...] = nxt
    z = (nxt - mean) / jnp.float32(SIGMA)
    tz = (-0.5 * z * z - jnp.float32(math.log(SIGMA))) - jnp.float32(_HALF_LOG_2PI)
    t = jax.lax.dot(tz, m16_ref[...], precision=jax.lax.Precision.HIGHEST)
    em = jax.lax.dot(nxt, cbd_ref[...], precision=jax.lax.Precision.HIGHEST)
    ze = obs_ref[0:1, :] - em
    se = (-0.5 * ze * ze) - jnp.float32(_HALF_LOG_2PI)
    e = jax.lax.dot(se, m8_ref[...], precision=jax.lax.Precision.HIGHEST)
    inc = (t + e) - t
    inc_ref[...] = inc
    lwr = jnp.where(flag, jnp.full_like(inc, jnp.float32(-math.log(N))),
                    lw_ref[...])
    newlw_ref[...] = lwr + inc


@jax.jit
def kernel(log_w, particles, observation, A, C):
    key = jax.random.key(42)
    rk, pk = jax.random.split(key)
    u0 = jax.random.uniform(rk, (), dtype=jnp.float32)
    eps_pack = jax.random.normal(pk, (RP, PACK * D), dtype=jnp.float32)

    lw = log_w - jax.scipy.special.logsumexp(log_w)
    w = jnp.exp(lw)
    cum = jnp.cumsum(w)

    smem_scalar = pl.BlockSpec(memory_space=pltpu.SMEM)
    blk = pl.BlockSpec((_AR, 128), lambda i: (i, 0))
    one_spec = pl.BlockSpec((1, 1), lambda i: (0, 0))
    f2d, ess11, flag11 = pl.pallas_call(
        _fcount_body,
        grid=(_AGRID,),
        in_specs=[smem_scalar, blk, blk],
        out_specs=[blk, one_spec, one_spec],
        out_shape=[
            jax.ShapeDtypeStruct((N // 128, 128), jnp.int32),
            jax.ShapeDtypeStruct((1, 1), jnp.float32),
            jax.ShapeDtypeStruct((1, 1), jnp.int32),
        ],
        scratch_shapes=[pltpu.SMEM((1, 1), jnp.float32)],
    )(u0.reshape(1, 1), cum.reshape(N // 128, 128), w.reshape(N // 128, 128))

    h2 = _hist_sc(f2d.reshape(N),
                  jnp.zeros((_HPAD,), jnp.int32),
                  jnp.ones((_BW,), jnp.int32))

    u128 = jnp.triu(jnp.ones((128, 128), jnp.float32))
    l256 = jnp.tril(jnp.ones((_AR, _AR), jnp.float32), -1)
    anc2d = pl.pallas_call(
        _scan_body,
        grid=(_AGRID,),
        in_specs=[
            smem_scalar,
            pl.BlockSpec((2, _AR, 128), lambda i: (0, i, 0)),
            pl.BlockSpec((128, 128), lambda i: (0, 0)),
            pl.BlockSpec((_AR, _AR), lambda i: (0, 0)),
        ],
        out_specs=blk,
        out_shape=jax.ShapeDtypeStruct((N // 128, 128), jnp.int32),
        scratch_shapes=[pltpu.SMEM((1, 1), jnp.float32)],
    )(flag11, h2.reshape(2, N // 128, 128), u128, l256)

    resampled = _gather_sc(particles, anc2d.reshape(N))

    rs_pack = resampled.reshape(RP, PACK * D)
    lw_pack = lw.reshape(RP, PACK)

    eye8 = jnp.eye(PACK, dtype=jnp.float32)
    abd = jnp.kron(eye8, A.T)
    cbd = jnp.kron(eye8, C.T)
    m16 = jnp.kron(eye8, jnp.ones((D, 1), jnp.float32))
    m8 = jnp.kron(eye8, jnp.ones((DO, 1), jnp.float32))
    obs_t = jnp.tile(observation, (PACK,)).reshape(1, PACK * DO)
    obs_t = jnp.broadcast_to(obs_t, (8, PACK * DO))

    row_spec = pl.BlockSpec((_ROWS, PACK * D), lambda i: (i, 0))
    out8_spec = pl.BlockSpec((_ROWS, PACK), lambda i: (i, 0))
    const_spec = lambda r, c: pl.BlockSpec((r, c), lambda i: (0, 0))
    nxt_pack, newlw_pack, inc_pack = pl.pallas_call(
        _dense_body,
        grid=(_GRID,),
        in_specs=[
            smem_scalar,
            row_spec,
            row_spec,
            out8_spec,
            const_spec(PACK * D, PACK * D),
            const_spec(PACK * D, PACK * DO),
            const_spec(PACK * D, PACK),
            const_spec(PACK * DO, PACK),
            const_spec(8, PACK * DO),
        ],
        out_specs=[row_spec, out8_spec, out8_spec],
        out_shape=[
            jax.ShapeDtypeStruct((RP, PACK * D), jnp.float32),
            jax.ShapeDtypeStruct((RP, PACK), jnp.float32),
            jax.ShapeDtypeStruct((RP, PACK), jnp.float32),
        ],
    )(flag11, rs_pack, eps_pack, lw_pack, abd, cbd, m16, m8, obs_t)

    next_particles = nxt_pack.reshape(N, D)
    new_log_w = newlw_pack.reshape(N)
    inc_weight = inc_pack.reshape(N)
    ess = ess11.reshape(())
    ancestor_ix = anc2d.reshape(N)
    return (new_log_w, next_particles, ess, ancestor_ix, inc_weight)


# speedup vs baseline: 19.4661x; 1.0095x over previous
import functools
import math

import jax
import jax.numpy as jnp
from jax import lax
from jax.experimental import pallas as pl
from jax.experimental.pallas import tpu as pltpu
from jax.experimental.pallas import tpu_sc as plsc

N = 262144
D = 16
DO = 8
SIGMA = 0.5
TAU = 1.0
PACK = 8
RP = N // PACK

_ROWS = 2048
_GRID = RP // _ROWS

_AR = 256
_AGRID = (N // 128) // _AR

_NW = 32
_BW = N // _NW
_HPAD = N + 256
_HSLC = _HPAD // 16
_GSUB = 4096

_HALF_LOG_2PI = 0.5 * math.log(2.0 * math.pi)


def _fcount_body(u0_ref, cum_ref, w_ref, f_ref, ess_ref, flag_ref, acc_ref):
    i = pl.program_id(0)

    @pl.when(i == 0)
    def _():
        acc_ref[0, 0] = 0.0

    u0 = u0_ref[0, 0]
    cum = cum_ref[...]
    t = cum * jnp.float32(N)
    i0 = jnp.floor(t - u0)
    cnt = i0 - 1.0
    for dc in (-1.0, 0.0, 1.0):
        cnt = cnt + jnp.where((i0 + jnp.float32(dc)) + u0 <= t, 1.0, 0.0)
    f_ref[...] = jnp.clip(cnt, 0.0, jnp.float32(N)).astype(jnp.int32)

    w = w_ref[...]
    acc_ref[0, 0] += jnp.sum(w * w)

    @pl.when(i == _AGRID - 1)
    def _():
        ess = 1.0 / acc_ref[0, 0]
        ess_ref[...] = jnp.full((1, 1), ess, jnp.float32)
        flag_ref[...] = jnp.where(ess < jnp.float32(0.5 * N),
                                  jnp.full((1, 1), 1, jnp.int32),
                                  jnp.full((1, 1), 0, jnp.int32))


_sc_mesh = plsc.VectorSubcoreMesh(core_axis_name="c", subcore_axis_name="s")


@functools.partial(
    pl.kernel, mesh=_sc_mesh,
    out_type=jax.ShapeDtypeStruct((2, N), jnp.int32),
    scratch_types=[
        pltpu.VMEM((_BW,), jnp.int32),
        pltpu.VMEM((_BW,), jnp.int32),
        pltpu.VMEM((_HSLC,), jnp.int32),
        pltpu.VMEM_SHARED((_HPAD,), jnp.int32),
    ],
)
def _hist_sc(f_hbm, zeros_hbm, ones_hbm, h_hbm, f_v, ones_v, vbuf, h_sh):
    cid = lax.axis_index("c")
    sid = lax.axis_index("s")
    wid = sid * 2 + cid
    pltpu.sync_copy(zeros_hbm.at[pl.ds(sid * _HSLC, _HSLC)], vbuf)
    pltpu.sync_copy(vbuf, h_sh.at[pl.ds(sid * _HSLC, _HSLC)])
    plsc.subcore_barrier()
    pltpu.sync_copy(f_hbm.at[pl.ds(wid * _BW, _BW)], f_v)
    pltpu.sync_copy(ones_hbm, ones_v)
    pltpu.sync_copy(ones_v, h_sh.at[f_v], add=True)
    plsc.subcore_barrier()
    pltpu.sync_copy(h_sh.at[pl.ds(sid * (N // 16), N // 16)],
                    vbuf.at[pl.ds(0, N // 16)])
    pltpu.sync_copy(vbuf.at[pl.ds(0, N // 16)],
                    h_hbm.at[cid, pl.ds(sid * (N // 16), N // 16)])


def _scan_body(flag_ref, h_ref, u128_ref, l256_ref, anc_ref, car_ref):
    i = pl.program_id(0)

    @pl.when(i == 0)
    def _():
        car_ref[0, 0] = 0.0

    x = (h_ref[0] + h_ref[1]).astype(jnp.float32)
    lane_pref = jax.lax.dot(x, u128_ref[...],
                            precision=jax.lax.Precision.HIGHEST)
    row_tot = jnp.broadcast_to(lane_pref[:, 127:128], (_AR, 128))
    row_off = jax.lax.dot(l256_ref[...], row_tot,
                          precision=jax.lax.Precision.HIGHEST)
    idxf = lane_pref + row_off + car_ref[0, 0]
    car_ref[0, 0] += jnp.sum(x)
    idx = jnp.clip(idxf, 0.0, jnp.float32(N - 1)).astype(jnp.int32)
    glob = (lax.broadcasted_iota(jnp.int32, (_AR, 128), 0) * 128
            + lax.broadcasted_iota(jnp.int32, (_AR, 128), 1)
            + i * (_AR * 128))
    anc_ref[...] = jnp.where(flag_ref[0, 0] > 0, idx, glob)


@functools.partial(
    pl.kernel, mesh=_sc_mesh,
    out_type=jax.ShapeDtypeStruct((N, D), jnp.float32),
    compiler_params=pltpu.CompilerParams(use_tc_tiling_on_sc=False),
    scratch_types=[
        pltpu.VMEM((_GSUB,), jnp.int32),
        pltpu.VMEM((_GSUB, D), jnp.float32),
        pltpu.SemaphoreType.DMA,
    ],
)
def _gather_sc(table_hbm, idx_hbm, out_hbm, idx_v, rows_v, sem):
    cid = lax.axis_index("c")
    sid = lax.axis_index("s")
    wid = sid * 2 + cid
    for g in range(_BW // _GSUB):
        base = wid * _BW + g * _GSUB
        pltpu.sync_copy(idx_hbm.at[pl.ds(base, _GSUB)], idx_v)
        pltpu.async_copy(table_hbm.at[idx_v], rows_v, sem).wait()
        pltpu.sync_copy(rows_v, out_hbm.at[pl.ds(base, _GSUB)])


def _dense_body(flag_ref, rs_ref, eps_ref, lw_ref, abd_ref, cbd_ref,
                m16_ref, m8_ref, obs_ref, nxt_ref, newlw_ref, inc_ref):
    flag = flag_ref[0, 0] > 0
    pr = rs_ref[...]
    mean = jax.lax.dot(pr, abd_ref[...],
                       precision=jax.lax.Precision.HIGHEST)
    nxt = mean + jnp.float32(SIGMA) * eps_ref[...]
    nxt_ref[...] = nxt
    z = (nxt - mean) / jnp.float32(SIGMA)
    tz = (-0.5 * z * z - jnp.float32(math.log(SIGMA))) - jnp.float32(_HALF_LOG_2PI)
    t = jax.lax.dot(tz, m16_ref[...], precision=jax.lax.Precision.HIGHEST)
    em = jax.lax.dot(nxt, cbd_ref[...], precision=jax.lax.Precision.HIGHEST)
    ze = obs_ref[0:1, :] - em
    se = (-0.5 * ze * ze) - jnp.float32(_HALF_LOG_2PI)
    e = jax.lax.dot(se, m8_ref[...], precision=jax.lax.Precision.HIGHEST)
    inc = (t + e) - t
    inc_ref[...] = inc
    lwr = jnp.where(flag, jnp.full_like(inc, jnp.float32(-math.log(N))),
                    lw_ref[...])
    newlw_ref[...] = lwr + inc


@jax.jit
def kernel(log_w, particles, observation, A, C):
    key = jax.random.key(42)
    rk, pk = jax.random.split(key)
    u0 = jax.random.uniform(rk, (), dtype=jnp.float32)
    eps_pack = jax.random.normal(pk, (RP, PACK * D), dtype=jnp.float32)

    lw = log_w - jax.scipy.special.logsumexp(log_w)
    w = jnp.exp(lw)
    cum = jnp.cumsum(w)

    smem_scalar = pl.BlockSpec(memory_space=pltpu.SMEM)
    blk = pl.BlockSpec((_AR, 128), lambda i: (i, 0))
    one_spec = pl.BlockSpec((1, 1), lambda i: (0, 0))
    f2d, ess11, flag11 = pl.pallas_call(
        _fcount_body,
        grid=(_AGRID,),
        in_specs=[smem_scalar, blk, blk],
        out_specs=[blk, one_spec, one_spec],
        out_shape=[
            jax.ShapeDtypeStruct((N // 128, 128), jnp.int32),
            jax.ShapeDtypeStruct((1, 1), jnp.float32),
            jax.ShapeDtypeStruct((1, 1), jnp.int32),
        ],
        scratch_shapes=[pltpu.SMEM((1, 1), jnp.float32)],
    )(u0.reshape(1, 1), cum.reshape(N // 128, 128), w.reshape(N // 128, 128))

    h2 = _hist_sc(f2d.reshape(N),
                  jnp.zeros((_HPAD,), jnp.int32),
                  jnp.ones((_BW,), jnp.int32))

    u128 = jnp.triu(jnp.ones((128, 128), jnp.float32))
    l256 = jnp.tril(jnp.ones((_AR, _AR), jnp.float32), -1)
    anc2d = pl.pallas_call(
        _scan_body,
        grid=(_AGRID,),
        in_specs=[
            smem_scalar,
            pl.BlockSpec((2, _AR, 128), lambda i: (0, i, 0)),
            pl.BlockSpec((128, 128), lambda i: (0, 0)),
            pl.BlockSpec((_AR, _AR), lambda i: (0, 0)),
        ],
        out_specs=blk,
        out_shape=jax.ShapeDtypeStruct((N // 128, 128), jnp.int32),
        scratch_shapes=[pltpu.SMEM((1, 1), jnp.float32)],
    )(flag11, h2.reshape(2, N // 128, 128), u128, l256)

    resampled = _gather_sc(particles, anc2d.reshape(N))

    rs_pack = resampled.reshape(RP, PACK * D)
    lw_pack = lw.reshape(RP, PACK)

    eye8 = jnp.eye(PACK, dtype=jnp.float32)
    abd = jnp.kron(eye8, A.T)
    cbd = jnp.kron(eye8, C.T)
    m16 = jnp.kron(eye8, jnp.ones((D, 1), jnp.float32))
    m8 = jnp.kron(eye8, jnp.ones((DO, 1), jnp.float32))
    obs_t = jnp.tile(observation, (PACK,)).reshape(1, PACK * DO)
    obs_t = jnp.broadcast_to(obs_t, (8, PACK * DO))

    row_spec = pl.BlockSpec((_ROWS, PACK * D), lambda i: (i, 0))
    out8_spec = pl.BlockSpec((_ROWS, PACK), lambda i: (i, 0))
    const_spec = lambda r, c: pl.BlockSpec((r, c), lambda i: (0, 0))
    nxt_pack, newlw_pack, inc_pack = pl.pallas_call(
        _dense_body,
        grid=(_GRID,),
        in_specs=[
            smem_scalar,
            row_spec,
            row_spec,
            out8_spec,
            const_spec(PACK * D, PACK * D),
            const_spec(PACK * D, PACK * DO),
            const_spec(PACK * D, PACK),
            const_spec(PACK * DO, PACK),
            const_spec(8, PACK * DO),
        ],
        out_specs=[row_spec, out8_spec, out8_spec],
        out_shape=[
            jax.ShapeDtypeStruct((RP, PACK * D), jnp.float32),
            jax.ShapeDtypeStruct((RP, PACK), jnp.float32),
            jax.ShapeDtypeStruct((RP, PACK), jnp.float32),
        ],
    )(flag11, rs_pack, eps_pack, lw_pack, abd, cbd, m16, m8, obs_t)

    next_particles = nxt_pack.reshape(N, D)
    new_log_w = newlw_pack.reshape(N)
    inc_weight = inc_pack.reshape(N)
    ess = ess11.reshape(())
    ancestor_ix = anc2d.reshape(N)
    return (new_log_w, next_particles, ess, ancestor_ix, inc_weight)
